# baseline (speedup 1.0000x reference)
import functools

import numpy as np

import jax
import jax.numpy as jnp
from jax import lax
from jax.experimental import pallas as pl
from jax.experimental.pallas import tpu as pltpu
from jax.experimental.pallas import tpu_sc as plsc

N_NODES = 10000
D = 128
E = 320000

NC = 2
NS = 16
LANES = 16
N_TILES = NC * NS

CHUNK = 80
NBUF = 3
RPC = 3
R0 = 42
R1 = 42
EPR = RPC * CHUNK
E_PAD = NS * (R0 + R1) * EPR

TILE_ROWS = 632

_PERM = np.stack([
    np.arange(4 * 32).reshape(4, 2, 16)[:, 0],
    np.arange(4 * 32).reshape(4, 2, 16)[:, 1],
], axis=-1).reshape(-1)


def _sc_body(x_hbm, src_hbm, dst_hbm, ew_hbm, out_hbm,
             srcw, dstw, eww, gb0, gb1, gb2, sb0, sb1, sb2,
             g0, g1, g2, s0, s1, s2, w0, w1, agg_sh):
    c = lax.axis_index("c")
    s = lax.axis_index("s")
    wid = c * NS + s
    gbuf = (gb0, gb1, gb2)
    sbuf = (sb0, sb1, sb2)
    gsem = (g0, g1, g2)
    ssem = (s0, s1, s2)
    wsem = (w0, w1)

    def _zrow(i, carry):
        for k in range(D // LANES):
            sb0[i, pl.ds(k * LANES, LANES)] = jnp.zeros((LANES,), jnp.float32)
        return carry
    lax.fori_loop(0, CHUNK, _zrow, 0)

    zbase = jnp.minimum(s * TILE_ROWS, N_NODES - TILE_ROWS)
    nfull = TILE_ROWS // CHUNK
    rem = TILE_ROWS - nfull * CHUNK
    for t in range(nfull):
        pltpu.sync_copy(sb0, agg_sh.at[pl.ds(zbase + t * CHUNK, CHUNK)])
    if rem:
        pltpu.sync_copy(sb0.at[pl.ds(0, rem)],
                        agg_sh.at[pl.ds(zbase + nfull * CHUNK, rem)])
    plsc.subcore_barrier()

    def load_window(r, p):
        pltpu.async_copy(src_hbm.at[wid, r], srcw.at[p], wsem[p])
        pltpu.async_copy(dst_hbm.at[wid, r], dstw.at[p], wsem[p])
        pltpu.async_copy(ew_hbm.at[wid, r], eww.at[p], wsem[p])

    def wait_window(p):
        for _ in range(3):
            pltpu.make_async_copy(src_hbm.at[0, 0], srcw.at[p], wsem[p]).wait()

    def start_gather(p, q, b):
        pltpu.async_copy(x_hbm.at[srcw.at[p, q]], gbuf[b], gsem[b])

    def wait_gather(b):
        pltpu.make_async_copy(x_hbm.at[pl.ds(0, CHUNK)], gbuf[b], gsem[b]).wait()

    def start_scatter(p, q, b):
        pltpu.async_copy(sbuf[b], agg_sh.at[dstw.at[p, q]], ssem[b], add=True)

    def wait_scatter(b):
        pltpu.make_async_copy(sbuf[b], agg_sh.at[pl.ds(0, CHUNK)], ssem[b]).wait()

    def scale(p, q, b):
        gb = gbuf[b]
        sb = sbuf[b]

        def _edge16(g, carry):
            e0 = g * LANES
            w16 = eww[p, q, pl.ds(e0, LANES)]
            for i in range(LANES):
                w = w16[i]
                for k in range(D // (2 * LANES)):
                    vi = gb[e0 + i, pl.ds(k * LANES, LANES)]
                    lo = plsc.bitcast(vi << 16, jnp.float32)
                    hi = plsc.bitcast(vi & jnp.int32(-65536), jnp.float32)
                    sb[e0 + i, pl.ds(k * 2 * LANES, LANES)] = lo * w
                    sb[e0 + i, pl.ds(k * 2 * LANES + LANES, LANES)] = hi * w
            return carry
        lax.fori_loop(0, CHUNK // LANES, _edge16, 0)

    def step(p, q, do_swait, nxt, wwait_p=None):
        b = q
        if do_swait:
            wait_scatter((b + 1) % NBUF)
        if wwait_p is not None:
            wait_window(wwait_p)
        if nxt is not None:
            start_gather(nxt[0], nxt[1], (b + 1) % NBUF)
        wait_gather(b)
        scale(p, q, b)
        start_scatter(p, q, b)

    def round_(r, p, first=False, last=False):
        step(p, 0, not first, (p, 1))
        step(p, 1, not first, (p, 2))
        if not last:
            load_window(r + 1, 1 - p)
            step(p, 2, True, (1 - p, 0), wwait_p=1 - p)
        else:
            step(p, 2, True, None)

    nr = jnp.where(c == 0, R0, R1)

    load_window(0, 0)
    wait_window(0)
    load_window(1, 1)
    start_gather(0, 0, 0)

    step(0, 0, False, (0, 1))
    step(0, 1, False, (0, 2))
    step(0, 2, True, (1, 0), wwait_p=1)
    round_(1, 1)

    def _dbl(dr, carry):
        r = dr * 2
        round_(r, 0)
        round_(r + 1, 1)
        return carry
    lax.fori_loop(1, nr // 2 - 1, _dbl, 0)

    round_(nr - 2, 0)
    round_(nr - 1, 1, last=True)
    wait_scatter(1)
    wait_scatter(2)
    plsc.subcore_barrier()

    pltpu.sync_copy(agg_sh.at[pl.ds(zbase, TILE_ROWS)],
                    out_hbm.at[c, pl.ds(zbase, TILE_ROWS)])


@functools.partial(
    pl.kernel,
    out_type=jax.ShapeDtypeStruct((NC, N_NODES, D), jnp.float32),
    mesh=plsc.VectorSubcoreMesh(core_axis_name="c", subcore_axis_name="s",
                                num_cores=NC, num_subcores=NS),
    compiler_params=pltpu.CompilerParams(use_tc_tiling_on_sc=False, needs_layout_passes=False),
    scratch_types=[
        pltpu.VMEM((2, RPC, CHUNK), jnp.int32),
        pltpu.VMEM((2, RPC, CHUNK), jnp.int32),
        pltpu.VMEM((2, RPC, CHUNK), jnp.float32),
        pltpu.VMEM((CHUNK, D // 2), jnp.int32),
        pltpu.VMEM((CHUNK, D // 2), jnp.int32),
        pltpu.VMEM((CHUNK, D // 2), jnp.int32),
        pltpu.VMEM((CHUNK, D), jnp.float32),
        pltpu.VMEM((CHUNK, D), jnp.float32),
        pltpu.VMEM((CHUNK, D), jnp.float32),
        pltpu.SemaphoreType.DMA,
        pltpu.SemaphoreType.DMA,
        pltpu.SemaphoreType.DMA,
        pltpu.SemaphoreType.DMA,
        pltpu.SemaphoreType.DMA,
        pltpu.SemaphoreType.DMA,
        pltpu.SemaphoreType.DMA,
        pltpu.SemaphoreType.DMA,
        pltpu.VMEM_SHARED((N_NODES, D), jnp.float32),
    ],
)
def _sc_scatter(x_hbm, src_hbm, dst_hbm, ew_hbm, out_hbm,
                srcw, dstw, eww, gb0, gb1, gb2, sb0, sb1, sb2,
                g0, g1, g2, s0, s1, s2, w0, w1, agg_sh):
    _sc_body(x_hbm, src_hbm, dst_hbm, ew_hbm, out_hbm,
             srcw, dstw, eww, gb0, gb1, gb2, sb0, sb1, sb2,
             g0, g1, g2, s0, s1, s2, w0, w1, agg_sh)


ROW_BLK = 2000


def _root_body(x_ref, wroot_ref, brel_ref, z_ref):
    dn = (((1,), (1,)), ((), ()))
    z_ref[...] = lax.dot_general(
        x_ref[...], wroot_ref[...], dn, preferred_element_type=jnp.float32
    ) + brel_ref[...]


def _tc_root(x, W_root, b_rel):
    grid = (N_NODES // ROW_BLK,)
    blk = pl.BlockSpec((ROW_BLK, D), lambda i: (i, 0))
    wblk = pl.BlockSpec((D, D), lambda i: (0, 0))
    bblk = pl.BlockSpec((1, D), lambda i: (0, 0))
    return pl.pallas_call(
        _root_body,
        grid=grid,
        in_specs=[blk, wblk, bblk],
        out_specs=blk,
        out_shape=jax.ShapeDtypeStruct((N_NODES, D), jnp.float32),
    )(x, W_root, b_rel.reshape(1, D))


def _rel_body(agg0_ref, agg1_ref, z_ref, wrel_ref, out_ref):
    a = agg0_ref[...] + agg1_ref[...]
    dn = (((1,), (1,)), ((), ()))
    out_ref[...] = lax.dot_general(
        a, wrel_ref[...], dn, preferred_element_type=jnp.float32
    ) + z_ref[...]


def _tc_rel(agg0, agg1, z, W_rel):
    grid = (N_NODES // ROW_BLK,)
    blk = pl.BlockSpec((ROW_BLK, D), lambda i: (i, 0))
    wblk = pl.BlockSpec((D, D), lambda i: (0, 0))
    return pl.pallas_call(
        _rel_body,
        grid=grid,
        in_specs=[blk, blk, blk, wblk],
        out_specs=blk,
        out_shape=jax.ShapeDtypeStruct((N_NODES, D), jnp.float32),
    )(agg0, agg1, z, W_rel)


def kernel(x, edge_index, edge_weight, W_rel, b_rel, W_root):
    src = edge_index[0].astype(jnp.int32)
    dst = edge_index[1].astype(jnp.int32)
    ew = edge_weight.astype(jnp.float32)

    x_bf = x.astype(jnp.bfloat16)[:, _PERM]
    x32 = lax.bitcast_convert_type(
        x_bf.reshape(N_NODES, D // 2, 2), jnp.int32)

    pad = E_PAD - src.shape[0]
    if pad:
        src = jnp.concatenate([src, jnp.zeros((pad,), jnp.int32)])
        dst = jnp.concatenate([dst, jnp.zeros((pad,), jnp.int32)])
        ew = jnp.concatenate([ew, jnp.zeros((pad,), jnp.float32)])

    def _tile_layout(v):
        if R0 == R1:
            return v.reshape(N_TILES, R0, RPC, CHUNK)
        cut = NS * R0 * EPR
        a = v[:cut].reshape(NS, R0, RPC, CHUNK)
        b = v[cut:].reshape(NS, R1, RPC, CHUNK)
        b = jnp.pad(b, ((0, 0), (0, R0 - R1), (0, 0), (0, 0)))
        return jnp.concatenate([a, b], axis=0)

    src4 = _tile_layout(src)
    dst4 = _tile_layout(dst)
    ew4 = _tile_layout(ew)

    z = _tc_root(x, W_root, b_rel)
    agg = _sc_scatter(x32, src4, dst4, ew4)
    return _tc_rel(agg[0], agg[1], z, W_rel)


# speedup vs baseline: 1.3374x; 1.3374x over previous
import functools

import jax
import jax.numpy as jnp
from jax import lax
from jax.experimental import pallas as pl
from jax.experimental.pallas import tpu as pltpu
from jax.experimental.pallas import tpu_sc as plsc

N_NODES = 10000
D = 128
E = 320000

NC = 2
NS = 16
LANES = 16
N_TILES = NC * NS

CHUNK = 112
NBUF = 3
RPC = 3
R0 = 34
R1 = 26
EPR = RPC * CHUNK
N_ROUNDS_ALL = NS * (R0 + R1)
E_PAD = N_ROUNDS_ALL * EPR

TILE_ROWS = 632


def _sc_body(x_hbm, src_hbm, dst_hbm, ew_hbm, out_hbm,
             srcw, dstw, eww, rows0, rows1, rows2,
             g0, g1, g2, s0, s1, s2, w0, w1, agg_sh):
    c = lax.axis_index("c")
    s = lax.axis_index("s")
    rows = (rows0, rows1, rows2)
    gsem = (g0, g1, g2)
    ssem = (s0, s1, s2)
    wsem = (w0, w1)

    rbase = jnp.where(c == 0, s * R0, NS * R0 + s * R1)
    nr = jnp.where(c == 0, R0, R1)

    def _zrow(i, carry):
        for k in range(D // LANES):
            rows0[i, pl.ds(k * LANES, LANES)] = jnp.zeros((LANES,), jnp.float32)
        return carry
    lax.fori_loop(0, CHUNK, _zrow, 0)

    zbase = jnp.minimum(s * TILE_ROWS, N_NODES - TILE_ROWS)
    nfull = TILE_ROWS // CHUNK
    rem = TILE_ROWS - nfull * CHUNK
    for t in range(nfull):
        pltpu.sync_copy(rows0, agg_sh.at[pl.ds(zbase + t * CHUNK, CHUNK)])
    if rem:
        pltpu.sync_copy(rows0.at[pl.ds(0, rem)],
                        agg_sh.at[pl.ds(zbase + nfull * CHUNK, rem)])
    plsc.subcore_barrier()

    def load_window(r, p):
        pltpu.async_copy(src_hbm.at[rbase + r], srcw.at[p], wsem[p])
        pltpu.async_copy(dst_hbm.at[rbase + r], dstw.at[p], wsem[p])
        pltpu.async_copy(ew_hbm.at[rbase + r], eww.at[p], wsem[p])

    def wait_window(p):
        for _ in range(3):
            pltpu.make_async_copy(src_hbm.at[0], srcw.at[p], wsem[p]).wait()

    def start_gather(p, q, b):
        pltpu.async_copy(x_hbm.at[srcw.at[p, q]], rows[b], gsem[b])

    def wait_gather(b):
        pltpu.make_async_copy(x_hbm.at[pl.ds(0, CHUNK)], rows[b], gsem[b]).wait()

    def start_scatter(p, q, b):
        pltpu.async_copy(rows[b], agg_sh.at[dstw.at[p, q]], ssem[b], add=True)

    def wait_scatter(b):
        pltpu.make_async_copy(rows[b], agg_sh.at[pl.ds(0, CHUNK)], ssem[b]).wait()

    def scale(p, q, b):
        buf = rows[b]

        def _edge16(g, carry):
            e0 = g * LANES
            w16 = eww[p, q, pl.ds(e0, LANES)]
            for i in range(LANES):
                w = w16[i]
                for k in range(D // LANES):
                    sl = pl.ds(k * LANES, LANES)
                    buf[e0 + i, sl] = buf[e0 + i, sl] * w
            return carry
        lax.fori_loop(0, CHUNK // LANES, _edge16, 0)

    def step(p, q, do_swait, nxt, wwait_p=None):
        b = q
        if do_swait:
            wait_scatter((b + 1) % NBUF)
        if wwait_p is not None:
            wait_window(wwait_p)
        if nxt is not None:
            start_gather(nxt[0], nxt[1], (b + 1) % NBUF)
        wait_gather(b)
        scale(p, q, b)
        start_scatter(p, q, b)

    def round_(r, p, first=False, last=False):
        step(p, 0, not first, (p, 1))
        step(p, 1, not first, (p, 2))
        if not last:
            load_window(r + 1, 1 - p)
            step(p, 2, True, (1 - p, 0), wwait_p=1 - p)
        else:
            step(p, 2, True, None)

    load_window(0, 0)
    wait_window(0)
    load_window(1, 1)
    start_gather(0, 0, 0)

    step(0, 0, False, (0, 1))
    step(0, 1, False, (0, 2))
    step(0, 2, True, (1, 0), wwait_p=1)
    round_(1, 1)

    def _dbl(dr, carry):
        r = dr * 2
        round_(r, 0)
        round_(r + 1, 1)
        return carry
    lax.fori_loop(1, nr // 2 - 1, _dbl, 0)

    round_(nr - 2, 0)
    round_(nr - 1, 1, last=True)
    wait_scatter(1)
    wait_scatter(2)
    plsc.subcore_barrier()

    pltpu.sync_copy(agg_sh.at[pl.ds(zbase, TILE_ROWS)],
                    out_hbm.at[c, pl.ds(zbase, TILE_ROWS)])


@functools.partial(
    pl.kernel,
    out_type=jax.ShapeDtypeStruct((NC, N_NODES, D), jnp.float32),
    mesh=plsc.VectorSubcoreMesh(core_axis_name="c", subcore_axis_name="s",
                                num_cores=NC, num_subcores=NS),
    scratch_types=[
        pltpu.VMEM((2, RPC, CHUNK), jnp.int32),
        pltpu.VMEM((2, RPC, CHUNK), jnp.int32),
        pltpu.VMEM((2, RPC, CHUNK), jnp.float32),
        pltpu.VMEM((CHUNK, D), jnp.float32),
        pltpu.VMEM((CHUNK, D), jnp.float32),
        pltpu.VMEM((CHUNK, D), jnp.float32),
        pltpu.SemaphoreType.DMA,
        pltpu.SemaphoreType.DMA,
        pltpu.SemaphoreType.DMA,
        pltpu.SemaphoreType.DMA,
        pltpu.SemaphoreType.DMA,
        pltpu.SemaphoreType.DMA,
        pltpu.SemaphoreType.DMA,
        pltpu.SemaphoreType.DMA,
        pltpu.VMEM_SHARED((N_NODES, D), jnp.float32),
    ],
)
def _sc_scatter(x_hbm, src_hbm, dst_hbm, ew_hbm, out_hbm,
                srcw, dstw, eww, rows0, rows1, rows2,
                g0, g1, g2, s0, s1, s2, w0, w1, agg_sh):
    _sc_body(x_hbm, src_hbm, dst_hbm, ew_hbm, out_hbm,
             srcw, dstw, eww, rows0, rows1, rows2,
             g0, g1, g2, s0, s1, s2, w0, w1, agg_sh)


ROW_BLK = 2000


def _root_body(x_ref, wroot_ref, brel_ref, z_ref):
    dn = (((1,), (1,)), ((), ()))
    z_ref[...] = lax.dot_general(
        x_ref[...], wroot_ref[...], dn, preferred_element_type=jnp.float32
    ) + brel_ref[...]


def _tc_root(x, W_root, b_rel):
    grid = (N_NODES // ROW_BLK,)
    blk = pl.BlockSpec((ROW_BLK, D), lambda i: (i, 0))
    wblk = pl.BlockSpec((D, D), lambda i: (0, 0))
    bblk = pl.BlockSpec((1, D), lambda i: (0, 0))
    return pl.pallas_call(
        _root_body,
        grid=grid,
        in_specs=[blk, wblk, bblk],
        out_specs=blk,
        out_shape=jax.ShapeDtypeStruct((N_NODES, D), jnp.float32),
    )(x, W_root, b_rel.reshape(1, D))


def _rel_body(agg0_ref, agg1_ref, z_ref, wrel_ref, out_ref):
    a = agg0_ref[...] + agg1_ref[...]
    dn = (((1,), (1,)), ((), ()))
    out_ref[...] = lax.dot_general(
        a, wrel_ref[...], dn, preferred_element_type=jnp.float32
    ) + z_ref[...]


def _tc_rel(agg0, agg1, z, W_rel):
    grid = (N_NODES // ROW_BLK,)
    blk = pl.BlockSpec((ROW_BLK, D), lambda i: (i, 0))
    wblk = pl.BlockSpec((D, D), lambda i: (0, 0))
    return pl.pallas_call(
        _rel_body,
        grid=grid,
        in_specs=[blk, blk, blk, wblk],
        out_specs=blk,
        out_shape=jax.ShapeDtypeStruct((N_NODES, D), jnp.float32),
    )(agg0, agg1, z, W_rel)


def kernel(x, edge_index, edge_weight, W_rel, b_rel, W_root):
    src = edge_index[0].astype(jnp.int32)
    dst = edge_index[1].astype(jnp.int32)
    ew = edge_weight.astype(jnp.float32)

    pad = E_PAD - src.shape[0]
    if pad:
        src = jnp.concatenate([src, jnp.zeros((pad,), jnp.int32)])
        dst = jnp.concatenate([dst, jnp.zeros((pad,), jnp.int32)])
        ew = jnp.concatenate([ew, jnp.zeros((pad,), jnp.float32)])

    src3 = src.reshape(N_ROUNDS_ALL, RPC, CHUNK)
    dst3 = dst.reshape(N_ROUNDS_ALL, RPC, CHUNK)
    ew3 = ew.reshape(N_ROUNDS_ALL, RPC, CHUNK)

    z = _tc_root(x, W_root, b_rel)
    agg = _sc_scatter(x, src3, dst3, ew3)
    return _tc_rel(agg[0], agg[1], z, W_rel)
